# baseline (speedup 1.0000x reference)
import math

import jax
import jax.numpy as jnp
from jax.experimental import pallas as pl
from jax.experimental.pallas import tpu as pltpu

D_MODEL = 384
D_HEAD = 64
N_HEAD = D_MODEL // D_HEAD
D_MID = D_MODEL * 4
B, H, W = 4, 32, 32
N_TOK = H * W
EPS = 1e-5
MID_CHUNK = 512
N_CHUNK = D_MID // MID_CHUNK

_F32 = jnp.float32
_BF16 = jnp.bfloat16


def _gelu(x):
    return 0.5 * x * (1.0 + jax.lax.erf(x * (1.0 / math.sqrt(2.0))))


def _block_body(x0_ref, qw_ref, kw_ref, vw_ref, mw_ref, vec_ref,
                w1_ref, w2_ref, dwt_ref, bmid_ref, out_ref):
    x0 = x0_ref[0]
    sc0 = vec_ref[:, 0:1]
    b0 = vec_ref[:, 1:2]
    mb = vec_ref[:, 2:3]
    b3 = vec_ref[:, 3:4]
    xg = _gelu(x0 * sc0 + b0).astype(_BF16)

    dn_cc = (((1,), (0,)), ((), ()))
    dn_tt = (((0,), (0,)), ((), ()))
    dn_nn = (((1,), (1,)), ((), ()))

    qT = jax.lax.dot_general(qw_ref[...], xg, dn_cc,
                             preferred_element_type=_F32).astype(_BF16)
    kT = jax.lax.dot_general(kw_ref[...], xg, dn_cc,
                             preferred_element_type=_F32).astype(_BF16)
    vT = jax.lax.dot_general(vw_ref[...], xg, dn_cc,
                             preferred_element_type=_F32).astype(_BF16)
    ones8 = jnp.ones((8, N_TOK), _BF16)

    acc = x0 + mb
    for h in range(N_HEAD):
        qh = jax.lax.slice(qT, (h * D_HEAD, 0), ((h + 1) * D_HEAD, N_TOK))
        kh = jax.lax.slice(kT, (h * D_HEAD, 0), ((h + 1) * D_HEAD, N_TOK))
        vh = jax.lax.slice(vT, (h * D_HEAD, 0), ((h + 1) * D_HEAD, N_TOK))
        s = jax.lax.dot_general(qh, kh, dn_tt, preferred_element_type=_F32)
        p = jnp.exp(jnp.minimum(s, 40.0)).astype(_BF16)
        vext = jnp.concatenate([vh, ones8], axis=0)
        mv = jax.lax.dot_general(p, vext, dn_nn,
                                 preferred_element_type=_F32)
        l = jax.lax.slice(mv, (0, D_HEAD), (N_TOK, D_HEAD + 1))
        mh = (jax.lax.slice(mv, (0, 0), (N_TOK, D_HEAD)) / l).astype(_BF16)
        acc = acc + jax.lax.dot_general(mw_ref[h], mh, dn_nn,
                                        preferred_element_type=_F32)

    xb = acc.astype(_BF16)

    col = jax.lax.broadcasted_iota(jnp.int32, (1, N_TOK), 1)
    w = col % W
    h_pos = col // W
    m_wl = w >= 1
    m_wr = w <= W - 2
    m_hu = h_pos >= 1
    m_hd = h_pos <= H - 2
    masks = {(-1, -1): m_hu & m_wl, (-1, 0): m_hu, (-1, 1): m_hu & m_wr,
             (0, -1): m_wl, (0, 0): None, (0, 1): m_wr,
             (1, -1): m_hd & m_wl, (1, 0): m_hd, (1, 1): m_hd & m_wr}

    acc2 = acc + b3
    for nc in range(N_CHUNK):
        w1c = w1_ref[nc * MID_CHUNK:(nc + 1) * MID_CHUNK, :]
        y = _gelu(jax.lax.dot_general(w1c, xb, dn_cc,
                                      preferred_element_type=_F32)
                  + bmid_ref[nc * MID_CHUNK:(nc + 1) * MID_CHUNK, 0:1])
        z = jnp.zeros((MID_CHUNK, N_TOK), _F32)
        idx = 0
        for dh in (-1, 0, 1):
            for dw in (-1, 0, 1):
                shift = dh * W + dw
                rolled = pltpu.roll(y, (-shift) % N_TOK, 1) if shift else y
                m = masks[(dh, dw)]
                if m is not None:
                    rolled = jnp.where(m, rolled, 0.0)
                tap = dwt_ref[nc * MID_CHUNK:(nc + 1) * MID_CHUNK, idx:idx + 1]
                z = z + rolled * tap
                idx += 1
        z = _gelu(z + bmid_ref[nc * MID_CHUNK:(nc + 1) * MID_CHUNK, 1:2])
        w2c = w2_ref[:, nc * MID_CHUNK:(nc + 1) * MID_CHUNK]
        acc2 = acc2 + jax.lax.dot_general(w2c, z.astype(_BF16), dn_cc,
                                          preferred_element_type=_F32)
    out_ref[0] = acc2


@jax.jit
def kernel(x0, bn0_g, bn0_b, q_w, k_w, v_w, merge_w, merge_b,
           mlp_w1, mlp_bn1_g, mlp_bn1_b, mlp_dw, mlp_bn2_g, mlp_bn2_b,
           mlp_w2, mlp_bn3_g, mlp_bn3_b):
    inv = 1.0 / math.sqrt(1.0 + EPS)
    x0c = x0.reshape(B, D_MODEL, N_TOK)

    qws = (q_w * (1.0 / math.sqrt(D_HEAD))).astype(_BF16)
    kwb = k_w.astype(_BF16)
    vwb = v_w.astype(_BF16)
    mw3 = merge_w.reshape(D_MODEL, N_HEAD, D_HEAD).transpose(1, 0, 2).astype(_BF16)

    vec = jnp.stack([bn0_g * inv, bn0_b, merge_b, mlp_bn3_b] +
                    [jnp.zeros((D_MODEL,), _F32)] * 4, axis=1)

    w1f = (mlp_w1 * (mlp_bn1_g * inv)[:, None]).astype(_BF16)
    w2f = (mlp_w2 * (mlp_bn3_g * inv)[:, None]).astype(_BF16)
    dwt = jnp.zeros((D_MID, 16), _F32)
    dwt = dwt.at[:, :9].set(mlp_dw.reshape(D_MID, 9)
                            * (mlp_bn2_g * inv)[:, None])
    bmid = jnp.stack([mlp_bn1_b, mlp_bn2_b] +
                     [jnp.zeros((D_MID,), _F32)] * 6, axis=1)

    out = pl.pallas_call(
        _block_body,
        grid=(B,),
        in_specs=[
            pl.BlockSpec((1, D_MODEL, N_TOK), lambda b: (b, 0, 0)),
            pl.BlockSpec((D_MODEL, D_MODEL), lambda b: (0, 0)),
            pl.BlockSpec((D_MODEL, D_MODEL), lambda b: (0, 0)),
            pl.BlockSpec((D_MODEL, D_MODEL), lambda b: (0, 0)),
            pl.BlockSpec((N_HEAD, D_MODEL, D_HEAD), lambda b: (0, 0, 0)),
            pl.BlockSpec((D_MODEL, 8), lambda b: (0, 0)),
            pl.BlockSpec((D_MID, D_MODEL), lambda b: (0, 0)),
            pl.BlockSpec((D_MODEL, D_MID), lambda b: (0, 0)),
            pl.BlockSpec((D_MID, 16), lambda b: (0, 0)),
            pl.BlockSpec((D_MID, 8), lambda b: (0, 0)),
        ],
        out_specs=pl.BlockSpec((1, D_MODEL, N_TOK), lambda b: (b, 0, 0)),
        out_shape=jax.ShapeDtypeStruct((B, D_MODEL, N_TOK), _F32),
        compiler_params=pltpu.CompilerParams(
            dimension_semantics=("parallel",)),
    )(x0c, qws, kwb, vwb, mw3, vec, w1f, w2f, dwt, bmid)

    return out.reshape(B, D_MODEL, H, W)


# speedup vs baseline: 1.0953x; 1.0953x over previous
import math

import jax
import jax.numpy as jnp
from jax.experimental import pallas as pl
from jax.experimental.pallas import tpu as pltpu

D_MODEL = 384
D_HEAD = 64
N_HEAD = D_MODEL // D_HEAD
D_MID = D_MODEL * 4
B, H, W = 4, 32, 32
N_TOK = H * W
EPS = 1e-5
PAD = 40
MID_CHUNK = 512
N_CHUNK = D_MID // MID_CHUNK

_F32 = jnp.float32
_BF16 = jnp.bfloat16


def _gelu(x):
    return 0.5 * x * (1.0 + jax.lax.erf(x * (1.0 / math.sqrt(2.0))))


def _attn_body(x0_ref, qw_ref, kw_ref, vw_ref, mw_ref, vec_ref, out_ref):
    x0 = x0_ref[0]
    sc0 = vec_ref[:, 0:1]
    b0 = vec_ref[:, 1:2]
    mb = vec_ref[:, 2:3]
    xg = _gelu(x0 * sc0 + b0).astype(_BF16)

    dn_tc = (((0,), (1,)), ((), ()))
    dn_nn = (((1,), (1,)), ((), ()))
    ones = jnp.ones((N_TOK, D_HEAD), _BF16)

    acc = x0 + mb
    for h in range(N_HEAD):
        qh = jax.lax.dot_general(xg, qw_ref[h], dn_tc,
                                 preferred_element_type=_F32).astype(_BF16)
        kh = jax.lax.dot_general(xg, kw_ref[h], dn_tc,
                                 preferred_element_type=_F32).astype(_BF16)
        vh = jax.lax.dot_general(xg, vw_ref[h], dn_tc,
                                 preferred_element_type=_F32).astype(_BF16)
        s = jax.lax.dot_general(qh, kh, dn_nn, preferred_element_type=_F32)
        p = jnp.exp(jnp.minimum(s, 40.0)).astype(_BF16)
        vext = jnp.concatenate([vh, ones], axis=1)
        mv = jax.lax.dot_general(p, vext, (((1,), (0,)), ((), ())),
                                 preferred_element_type=_F32)
        l = jax.lax.slice(mv, (0, D_HEAD), (N_TOK, D_HEAD + 1))
        mh = (jax.lax.slice(mv, (0, 0), (N_TOK, D_HEAD)) / l).astype(_BF16)
        acc = acc + jax.lax.dot_general(mw_ref[h], mh, dn_nn,
                                        preferred_element_type=_F32)
    out_ref[0] = acc


def _mlp_body(x_ref, w1_ref, w2_ref, dwt_ref, bmid_ref, vec_ref, out_ref):
    nc = pl.program_id(1)
    xc = x_ref[0]
    y = _gelu(jax.lax.dot_general(xc.astype(_BF16), w1_ref[...],
                                  (((0,), (1,)), ((), ())),
                                  preferred_element_type=_F32)
              + bmid_ref[0:1, :])
    zp = jnp.concatenate(
        [jnp.zeros((PAD, MID_CHUNK), _F32), y, jnp.zeros((PAD, MID_CHUNK), _F32)],
        axis=0)
    col = jax.lax.broadcasted_iota(jnp.int32, (N_TOK, 1), 0) % W
    mask_l = col >= 1
    mask_r = col <= W - 2
    z = jnp.zeros((N_TOK, MID_CHUNK), _F32)
    idx = 0
    for dh in (-1, 0, 1):
        for dw in (-1, 0, 1):
            sl = jax.lax.slice(zp, (PAD + dh * W + dw, 0),
                               (PAD + dh * W + dw + N_TOK, MID_CHUNK))
            if dw == -1:
                sl = jnp.where(mask_l, sl, 0.0)
            elif dw == 1:
                sl = jnp.where(mask_r, sl, 0.0)
            z = z + sl * dwt_ref[idx:idx + 1, :]
            idx += 1
    z = _gelu(z + bmid_ref[1:2, :])
    part = jax.lax.dot_general(w2_ref[...], z.astype(_BF16), dn_nn := (((1,), (1,)), ((), ())),
                               preferred_element_type=_F32)

    @pl.when(nc == 0)
    def _():
        out_ref[0] = xc + vec_ref[:, 3:4] + part

    @pl.when(nc != 0)
    def _():
        out_ref[0] = out_ref[0] + part


@jax.jit
def kernel(x0, bn0_g, bn0_b, q_w, k_w, v_w, merge_w, merge_b,
           mlp_w1, mlp_bn1_g, mlp_bn1_b, mlp_dw, mlp_bn2_g, mlp_bn2_b,
           mlp_w2, mlp_bn3_g, mlp_bn3_b):
    inv = 1.0 / math.sqrt(1.0 + EPS)
    x0c = x0.reshape(B, D_MODEL, N_TOK)

    qw3 = (q_w * (1.0 / math.sqrt(D_HEAD))).reshape(
        N_HEAD, D_HEAD, D_MODEL).astype(_BF16)
    kw3 = k_w.reshape(N_HEAD, D_HEAD, D_MODEL).astype(_BF16)
    vw3 = v_w.reshape(N_HEAD, D_HEAD, D_MODEL).astype(_BF16)
    mw3 = merge_w.reshape(D_MODEL, N_HEAD, D_HEAD).transpose(1, 0, 2).astype(_BF16)

    vec = jnp.stack([bn0_g * inv, bn0_b, merge_b, mlp_bn3_b] +
                    [jnp.zeros((D_MODEL,), _F32)] * 4, axis=1)

    w1f = (mlp_w1 * (mlp_bn1_g * inv)[:, None]).astype(_BF16)
    w2f = (mlp_w2 * (mlp_bn3_g * inv)[:, None]).astype(_BF16)
    dwt = jnp.zeros((16, D_MID), _F32)
    dwt = dwt.at[:9].set((mlp_dw.reshape(D_MID, 9)
                          * (mlp_bn2_g * inv)[:, None]).T)
    bmid = jnp.zeros((8, D_MID), _F32)
    bmid = bmid.at[0].set(mlp_bn1_b).at[1].set(mlp_bn2_b)

    xmid = pl.pallas_call(
        _attn_body,
        grid=(B,),
        in_specs=[
            pl.BlockSpec((1, D_MODEL, N_TOK), lambda b: (b, 0, 0)),
            pl.BlockSpec((N_HEAD, D_HEAD, D_MODEL), lambda b: (0, 0, 0)),
            pl.BlockSpec((N_HEAD, D_HEAD, D_MODEL), lambda b: (0, 0, 0)),
            pl.BlockSpec((N_HEAD, D_HEAD, D_MODEL), lambda b: (0, 0, 0)),
            pl.BlockSpec((N_HEAD, D_MODEL, D_HEAD), lambda b: (0, 0, 0)),
            pl.BlockSpec((D_MODEL, 8), lambda b: (0, 0)),
        ],
        out_specs=pl.BlockSpec((1, D_MODEL, N_TOK), lambda b: (b, 0, 0)),
        out_shape=jax.ShapeDtypeStruct((B, D_MODEL, N_TOK), _F32),
        compiler_params=pltpu.CompilerParams(
            dimension_semantics=("parallel",)),
    )(x0c, qw3, kw3, vw3, mw3, vec)

    out = pl.pallas_call(
        _mlp_body,
        grid=(B, N_CHUNK),
        in_specs=[
            pl.BlockSpec((1, D_MODEL, N_TOK), lambda b, nc: (b, 0, 0)),
            pl.BlockSpec((MID_CHUNK, D_MODEL), lambda b, nc: (nc, 0)),
            pl.BlockSpec((D_MODEL, MID_CHUNK), lambda b, nc: (0, nc)),
            pl.BlockSpec((16, MID_CHUNK), lambda b, nc: (0, nc)),
            pl.BlockSpec((8, MID_CHUNK), lambda b, nc: (0, nc)),
            pl.BlockSpec((D_MODEL, 8), lambda b, nc: (0, 0)),
        ],
        out_specs=pl.BlockSpec((1, D_MODEL, N_TOK), lambda b, nc: (b, 0, 0)),
        out_shape=jax.ShapeDtypeStruct((B, D_MODEL, N_TOK), _F32),
        compiler_params=pltpu.CompilerParams(
            dimension_semantics=("parallel", "arbitrary")),
    )(xmid, w1f, w2f, dwt, bmid, vec)

    return out.reshape(B, D_MODEL, H, W)


# speedup vs baseline: 1.2149x; 1.1092x over previous
import math

import jax
import jax.numpy as jnp
from jax.experimental import pallas as pl
from jax.experimental.pallas import tpu as pltpu

D_MODEL = 384
D_HEAD = 64
N_HEAD = D_MODEL // D_HEAD
D_MID = D_MODEL * 4
B, H, W = 4, 32, 32
N_TOK = H * W
EPS = 1e-5
PAD = 40
MID_CHUNK = 512
N_CHUNK = D_MID // MID_CHUNK

_F32 = jnp.float32
_BF16 = jnp.bfloat16


def _gelu(x):
    return 0.5 * x * (1.0 + jax.lax.erf(x * (1.0 / math.sqrt(2.0))))


def _attn_body(x0_ref, qw_ref, kw_ref, vw_ref, mw_ref, vec_ref, out_ref):
    x0 = x0_ref[0]
    sc0 = vec_ref[0:1, :]
    b0 = vec_ref[1:2, :]
    mb = vec_ref[2:3, :]
    xg = _gelu(x0 * sc0 + b0).astype(_BF16)
    acc = x0 + mb
    ones = jnp.ones((N_TOK, D_HEAD), _BF16)
    dn_cc = (((1,), (1,)), ((), ()))
    for h in range(N_HEAD):
        qh = jax.lax.dot_general(xg, qw_ref[h], dn_cc,
                                 preferred_element_type=_F32).astype(_BF16)
        kh = jax.lax.dot_general(xg, kw_ref[h], dn_cc,
                                 preferred_element_type=_F32).astype(_BF16)
        vh = jax.lax.dot_general(xg, vw_ref[h], dn_cc,
                                 preferred_element_type=_F32).astype(_BF16)
        s = jax.lax.dot_general(qh, kh, dn_cc, preferred_element_type=_F32)
        p = jnp.exp(jnp.minimum(s, 40.0)).astype(_BF16)
        vext = jnp.concatenate([vh, ones], axis=1)
        mv = jax.lax.dot_general(p, vext, (((1,), (0,)), ((), ())),
                                 preferred_element_type=_F32)
        l = jax.lax.slice(mv, (0, D_HEAD), (N_TOK, D_HEAD + 1))
        mh = (jax.lax.slice(mv, (0, 0), (N_TOK, D_HEAD)) / l).astype(_BF16)
        acc = acc + jax.lax.dot_general(mh, mw_ref[h], dn_cc,
                                        preferred_element_type=_F32)
    out_ref[0] = acc


def _mlp_body(x_ref, w1_ref, w2_ref, dwt_ref, bmid_ref, b3_ref, out_ref):
    nc = pl.program_id(1)
    x = x_ref[0]
    dn_cc = (((1,), (1,)), ((), ()))
    y = _gelu(jax.lax.dot_general(x.astype(_BF16), w1_ref[...], dn_cc,
                                  preferred_element_type=_F32)
              + bmid_ref[0:1, :])
    z8 = jnp.zeros((8, MID_CHUNK), _F32)
    yp8 = jnp.concatenate([z8, y, z8], axis=0)
    col = jax.lax.broadcasted_iota(jnp.int32, (N_TOK, 1), 0) % W
    um = jnp.where(col >= 1,
                   jax.lax.slice(yp8, (7, 0), (7 + N_TOK, MID_CHUNK)), 0.0)
    up = jnp.where(col <= W - 2,
                   jax.lax.slice(yp8, (9, 0), (9 + N_TOK, MID_CHUNK)), 0.0)

    def rowconv(i):
        return (um * dwt_ref[3 * i:3 * i + 1, :]
                + y * dwt_ref[3 * i + 1:3 * i + 2, :]
                + up * dwt_ref[3 * i + 2:3 * i + 3, :])

    zW = jnp.zeros((W, MID_CHUNK), _F32)
    cm1p = jnp.concatenate([zW, rowconv(0), zW], axis=0)
    cp1p = jnp.concatenate([zW, rowconv(2), zW], axis=0)
    z = (rowconv(1)
         + jax.lax.slice(cm1p, (0, 0), (N_TOK, MID_CHUNK))
         + jax.lax.slice(cp1p, (2 * W, 0), (2 * W + N_TOK, MID_CHUNK)))
    z = _gelu(z + bmid_ref[1:2, :])
    part = jax.lax.dot_general(z.astype(_BF16), w2_ref[...], dn_cc,
                               preferred_element_type=_F32)

    @pl.when(nc == 0)
    def _():
        out_ref[0] = x + b3_ref[0:1, :] + part

    @pl.when(nc != 0)
    def _():
        out_ref[0] = out_ref[0] + part


@jax.jit
def kernel(x0, bn0_g, bn0_b, q_w, k_w, v_w, merge_w, merge_b,
           mlp_w1, mlp_bn1_g, mlp_bn1_b, mlp_dw, mlp_bn2_g, mlp_bn2_b,
           mlp_w2, mlp_bn3_g, mlp_bn3_b):
    inv = 1.0 / math.sqrt(1.0 + EPS)
    x0t = x0.reshape(B, D_MODEL, N_TOK).transpose(0, 2, 1)

    qw3 = (q_w * (1.0 / math.sqrt(D_HEAD))).reshape(
        N_HEAD, D_HEAD, D_MODEL).astype(_BF16)
    kw3 = k_w.reshape(N_HEAD, D_HEAD, D_MODEL).astype(_BF16)
    vw3 = v_w.reshape(N_HEAD, D_HEAD, D_MODEL).astype(_BF16)
    mw3 = merge_w.reshape(D_MODEL, N_HEAD, D_HEAD).transpose(1, 0, 2).astype(_BF16)

    vec1 = jnp.zeros((8, D_MODEL), _F32)
    vec1 = vec1.at[0].set(bn0_g * inv).at[1].set(bn0_b).at[2].set(merge_b)

    w1f = (mlp_w1 * (mlp_bn1_g * inv)[:, None]).astype(_BF16)
    w2f = (mlp_w2 * (mlp_bn3_g * inv)[:, None]).astype(_BF16)
    dwt = jnp.zeros((16, D_MID), _F32)
    dwt = dwt.at[:9].set((mlp_dw.reshape(D_MID, 9)
                          * (mlp_bn2_g * inv)[:, None]).T)
    bmid = jnp.zeros((8, D_MID), _F32)
    bmid = bmid.at[0].set(mlp_bn1_b).at[1].set(mlp_bn2_b)
    b3 = jnp.zeros((8, D_MODEL), _F32)
    b3 = b3.at[0].set(mlp_bn3_b)

    xmid = pl.pallas_call(
        _attn_body,
        grid=(B,),
        in_specs=[
            pl.BlockSpec((1, N_TOK, D_MODEL), lambda b: (b, 0, 0)),
            pl.BlockSpec((N_HEAD, D_HEAD, D_MODEL), lambda b: (0, 0, 0)),
            pl.BlockSpec((N_HEAD, D_HEAD, D_MODEL), lambda b: (0, 0, 0)),
            pl.BlockSpec((N_HEAD, D_HEAD, D_MODEL), lambda b: (0, 0, 0)),
            pl.BlockSpec((N_HEAD, D_MODEL, D_HEAD), lambda b: (0, 0, 0)),
            pl.BlockSpec((8, D_MODEL), lambda b: (0, 0)),
        ],
        out_specs=pl.BlockSpec((1, N_TOK, D_MODEL), lambda b: (b, 0, 0)),
        out_shape=jax.ShapeDtypeStruct((B, N_TOK, D_MODEL), _F32),
        compiler_params=pltpu.CompilerParams(
            dimension_semantics=("parallel",)),
    )(x0t, qw3, kw3, vw3, mw3, vec1)

    out = pl.pallas_call(
        _mlp_body,
        grid=(B, N_CHUNK),
        in_specs=[
            pl.BlockSpec((1, N_TOK, D_MODEL), lambda b, nc: (b, 0, 0)),
            pl.BlockSpec((MID_CHUNK, D_MODEL), lambda b, nc: (nc, 0)),
            pl.BlockSpec((D_MODEL, MID_CHUNK), lambda b, nc: (0, nc)),
            pl.BlockSpec((16, MID_CHUNK), lambda b, nc: (0, nc)),
            pl.BlockSpec((8, MID_CHUNK), lambda b, nc: (0, nc)),
            pl.BlockSpec((8, D_MODEL), lambda b, nc: (0, 0)),
        ],
        out_specs=pl.BlockSpec((1, N_TOK, D_MODEL), lambda b, nc: (b, 0, 0)),
        out_shape=jax.ShapeDtypeStruct((B, N_TOK, D_MODEL), _F32),
        compiler_params=pltpu.CompilerParams(
            dimension_semantics=("parallel", "arbitrary")),
    )(xmid, w1f, w2f, dwt, bmid, b3)

    return out.transpose(0, 2, 1).reshape(B, D_MODEL, H, W)


# speedup vs baseline: 1.5065x; 1.2400x over previous
import math

import jax
import jax.numpy as jnp
from jax.experimental import pallas as pl
from jax.experimental.pallas import tpu as pltpu

D_MODEL = 384
D_HEAD = 64
N_HEAD = D_MODEL // D_HEAD
D_MID = D_MODEL * 4
B, H, W = 4, 32, 32
N_TOK = H * W
EPS = 1e-5
PAD = 40
MID_CHUNK = 512
N_CHUNK = D_MID // MID_CHUNK

_F32 = jnp.float32
_BF16 = jnp.bfloat16


def _gelu(x):
    return 0.5 * x * (1.0 + jax.lax.erf(x * (1.0 / math.sqrt(2.0))))


def _attn_body(x0_ref, qw_ref, kw_ref, vw_ref, mw_ref, vec_ref, out_ref):
    x0 = x0_ref[0]
    sc0 = vec_ref[0:1, :]
    b0 = vec_ref[1:2, :]
    mb = vec_ref[2:3, :]
    xg = _gelu(x0 * sc0 + b0).astype(_BF16)
    ones = jnp.ones((N_TOK, D_HEAD), _BF16)
    dn_cc = (((1,), (1,)), ((), ()))
    qa = jax.lax.dot_general(xg, qw_ref[...], dn_cc,
                             preferred_element_type=_F32).astype(_BF16)
    ka = jax.lax.dot_general(xg, kw_ref[...], dn_cc,
                             preferred_element_type=_F32).astype(_BF16)
    va = jax.lax.dot_general(xg, vw_ref[...], dn_cc,
                             preferred_element_type=_F32).astype(_BF16)
    mhs = []
    for h in range(N_HEAD):
        qh = jax.lax.slice(qa, (0, h * D_HEAD), (N_TOK, (h + 1) * D_HEAD))
        kh = jax.lax.slice(ka, (0, h * D_HEAD), (N_TOK, (h + 1) * D_HEAD))
        vh = jax.lax.slice(va, (0, h * D_HEAD), (N_TOK, (h + 1) * D_HEAD))
        s = jax.lax.dot_general(qh, kh, dn_cc, preferred_element_type=_F32)
        p = jnp.exp(jnp.minimum(s, 40.0)).astype(_BF16)
        vext = jnp.concatenate([vh, ones], axis=1)
        mv = jax.lax.dot_general(p, vext, (((1,), (0,)), ((), ())),
                                 preferred_element_type=_F32)
        l = jax.lax.slice(mv, (0, D_HEAD), (N_TOK, D_HEAD + 1))
        mhs.append((jax.lax.slice(mv, (0, 0), (N_TOK, D_HEAD)) / l).astype(_BF16))
    msg = jnp.concatenate(mhs, axis=1)
    out_ref[0] = x0 + mb + jax.lax.dot_general(
        msg, mw_ref[...], dn_cc, preferred_element_type=_F32)


def _mlp_body(x_ref, w1_ref, w2_ref, dwt_ref, bmid_ref, b3_ref, out_ref):
    nc = pl.program_id(1)
    x = x_ref[0]
    dn_cc = (((1,), (1,)), ((), ()))
    y = _gelu(jax.lax.dot_general(x.astype(_BF16), w1_ref[...], dn_cc,
                                  preferred_element_type=_F32)
              + bmid_ref[0:1, :])
    z8 = jnp.zeros((8, MID_CHUNK), _F32)
    yp8 = jnp.concatenate([z8, y, z8], axis=0)
    col = jax.lax.broadcasted_iota(jnp.int32, (N_TOK, 1), 0) % W
    um = jnp.where(col >= 1,
                   jax.lax.slice(yp8, (7, 0), (7 + N_TOK, MID_CHUNK)), 0.0)
    up = jnp.where(col <= W - 2,
                   jax.lax.slice(yp8, (9, 0), (9 + N_TOK, MID_CHUNK)), 0.0)

    def rowconv(i):
        return (um * dwt_ref[3 * i:3 * i + 1, :]
                + y * dwt_ref[3 * i + 1:3 * i + 2, :]
                + up * dwt_ref[3 * i + 2:3 * i + 3, :])

    zW = jnp.zeros((W, MID_CHUNK), _F32)
    cm1p = jnp.concatenate([zW, rowconv(0), zW], axis=0)
    cp1p = jnp.concatenate([zW, rowconv(2), zW], axis=0)
    z = (rowconv(1)
         + jax.lax.slice(cm1p, (0, 0), (N_TOK, MID_CHUNK))
         + jax.lax.slice(cp1p, (2 * W, 0), (2 * W + N_TOK, MID_CHUNK)))
    z = _gelu(z + bmid_ref[1:2, :])
    part = jax.lax.dot_general(z.astype(_BF16), w2_ref[...], dn_cc,
                               preferred_element_type=_F32)

    @pl.when(nc == 0)
    def _():
        out_ref[0] = x + b3_ref[0:1, :] + part

    @pl.when(nc != 0)
    def _():
        out_ref[0] = out_ref[0] + part


@jax.jit
def kernel(x0, bn0_g, bn0_b, q_w, k_w, v_w, merge_w, merge_b,
           mlp_w1, mlp_bn1_g, mlp_bn1_b, mlp_dw, mlp_bn2_g, mlp_bn2_b,
           mlp_w2, mlp_bn3_g, mlp_bn3_b):
    inv = 1.0 / math.sqrt(1.0 + EPS)
    x0t = x0.reshape(B, D_MODEL, N_TOK).transpose(0, 2, 1)

    qw2 = (q_w * (1.0 / math.sqrt(D_HEAD))).astype(_BF16)
    kw2 = k_w.astype(_BF16)
    vw2 = v_w.astype(_BF16)
    mw2 = merge_w.astype(_BF16)

    vec1 = jnp.zeros((8, D_MODEL), _F32)
    vec1 = vec1.at[0].set(bn0_g * inv).at[1].set(bn0_b).at[2].set(merge_b)

    w1f = (mlp_w1 * (mlp_bn1_g * inv)[:, None]).astype(_BF16)
    w2f = (mlp_w2 * (mlp_bn3_g * inv)[:, None]).astype(_BF16)
    dwt = jnp.zeros((16, D_MID), _F32)
    dwt = dwt.at[:9].set((mlp_dw.reshape(D_MID, 9)
                          * (mlp_bn2_g * inv)[:, None]).T)
    bmid = jnp.zeros((8, D_MID), _F32)
    bmid = bmid.at[0].set(mlp_bn1_b).at[1].set(mlp_bn2_b)
    b3 = jnp.zeros((8, D_MODEL), _F32)
    b3 = b3.at[0].set(mlp_bn3_b)

    xmid = pl.pallas_call(
        _attn_body,
        grid=(B,),
        in_specs=[
            pl.BlockSpec((1, N_TOK, D_MODEL), lambda b: (b, 0, 0)),
            pl.BlockSpec((D_MODEL, D_MODEL), lambda b: (0, 0)),
            pl.BlockSpec((D_MODEL, D_MODEL), lambda b: (0, 0)),
            pl.BlockSpec((D_MODEL, D_MODEL), lambda b: (0, 0)),
            pl.BlockSpec((D_MODEL, D_MODEL), lambda b: (0, 0)),
            pl.BlockSpec((8, D_MODEL), lambda b: (0, 0)),
        ],
        out_specs=pl.BlockSpec((1, N_TOK, D_MODEL), lambda b: (b, 0, 0)),
        out_shape=jax.ShapeDtypeStruct((B, N_TOK, D_MODEL), _F32),
        compiler_params=pltpu.CompilerParams(
            dimension_semantics=("parallel",)),
    )(x0t, qw2, kw2, vw2, mw2, vec1)

    out = pl.pallas_call(
        _mlp_body,
        grid=(B, N_CHUNK),
        in_specs=[
            pl.BlockSpec((1, N_TOK, D_MODEL), lambda b, nc: (b, 0, 0)),
            pl.BlockSpec((MID_CHUNK, D_MODEL), lambda b, nc: (nc, 0)),
            pl.BlockSpec((D_MODEL, MID_CHUNK), lambda b, nc: (0, nc)),
            pl.BlockSpec((16, MID_CHUNK), lambda b, nc: (0, nc)),
            pl.BlockSpec((8, MID_CHUNK), lambda b, nc: (0, nc)),
            pl.BlockSpec((8, D_MODEL), lambda b, nc: (0, 0)),
        ],
        out_specs=pl.BlockSpec((1, N_TOK, D_MODEL), lambda b, nc: (b, 0, 0)),
        out_shape=jax.ShapeDtypeStruct((B, N_TOK, D_MODEL), _F32),
        compiler_params=pltpu.CompilerParams(
            dimension_semantics=("parallel", "arbitrary")),
    )(xmid, w1f, w2f, dwt, bmid, b3)

    return out.transpose(0, 2, 1).reshape(B, D_MODEL, H, W)


# speedup vs baseline: 1.5852x; 1.0522x over previous
import math

import jax
import jax.numpy as jnp
from jax.experimental import pallas as pl
from jax.experimental.pallas import tpu as pltpu

D_MODEL = 384
D_HEAD = 64
N_HEAD = D_MODEL // D_HEAD
D_MID = D_MODEL * 4
B, H, W = 4, 32, 32
N_TOK = H * W
EPS = 1e-5
PAD = 40
MID_CHUNK = 768
N_CHUNK = D_MID // MID_CHUNK

_F32 = jnp.float32
_BF16 = jnp.bfloat16


def _gelu(x):
    return 0.5 * x * (1.0 + jax.lax.erf(x * (1.0 / math.sqrt(2.0))))


def _attn_body(x0_ref, qw_ref, kw_ref, vw_ref, mw_ref, vec_ref, out_ref):
    x0 = x0_ref[0]
    sc0 = vec_ref[0:1, :]
    b0 = vec_ref[1:2, :]
    mb = vec_ref[2:3, :]
    xg = _gelu(x0 * sc0 + b0).astype(_BF16)
    ones = jnp.ones((N_TOK, D_HEAD), _BF16)
    dn_cc = (((1,), (1,)), ((), ()))
    qa = jax.lax.dot_general(xg, qw_ref[...], dn_cc,
                             preferred_element_type=_F32).astype(_BF16)
    ka = jax.lax.dot_general(xg, kw_ref[...], dn_cc,
                             preferred_element_type=_F32).astype(_BF16)
    va = jax.lax.dot_general(xg, vw_ref[...], dn_cc,
                             preferred_element_type=_F32).astype(_BF16)
    mhs = []
    for h in range(N_HEAD):
        qh = jax.lax.slice(qa, (0, h * D_HEAD), (N_TOK, (h + 1) * D_HEAD))
        kh = jax.lax.slice(ka, (0, h * D_HEAD), (N_TOK, (h + 1) * D_HEAD))
        vh = jax.lax.slice(va, (0, h * D_HEAD), (N_TOK, (h + 1) * D_HEAD))
        s = jax.lax.dot_general(qh, kh, dn_cc, preferred_element_type=_F32)
        p = jnp.exp(jnp.minimum(s, 40.0)).astype(_BF16)
        vext = jnp.concatenate([vh, ones], axis=1)
        mv = jax.lax.dot_general(p, vext, (((1,), (0,)), ((), ())),
                                 preferred_element_type=_F32)
        l = jax.lax.slice(mv, (0, D_HEAD), (N_TOK, D_HEAD + 1))
        mhs.append((jax.lax.slice(mv, (0, 0), (N_TOK, D_HEAD)) / l).astype(_BF16))
    msg = jnp.concatenate(mhs, axis=1)
    out_ref[0] = x0 + mb + jax.lax.dot_general(
        msg, mw_ref[...], dn_cc, preferred_element_type=_F32)


def _mlp_body(x_ref, w1_ref, w2_ref, dwt_ref, bmid_ref, b3_ref, out_ref):
    nc = pl.program_id(1)
    x = x_ref[0]
    dn_cc = (((1,), (1,)), ((), ()))
    y = _gelu(jax.lax.dot_general(x.astype(_BF16), w1_ref[...], dn_cc,
                                  preferred_element_type=_F32)
              + bmid_ref[0:1, :])
    z8 = jnp.zeros((8, MID_CHUNK), _F32)
    yp8 = jnp.concatenate([z8, y, z8], axis=0)
    col = jax.lax.broadcasted_iota(jnp.int32, (N_TOK, 1), 0) % W
    um = jnp.where(col >= 1,
                   jax.lax.slice(yp8, (7, 0), (7 + N_TOK, MID_CHUNK)), 0.0)
    up = jnp.where(col <= W - 2,
                   jax.lax.slice(yp8, (9, 0), (9 + N_TOK, MID_CHUNK)), 0.0)

    def rowconv(i):
        return (um * dwt_ref[3 * i:3 * i + 1, :]
                + y * dwt_ref[3 * i + 1:3 * i + 2, :]
                + up * dwt_ref[3 * i + 2:3 * i + 3, :])

    zW = jnp.zeros((W, MID_CHUNK), _F32)
    cm1p = jnp.concatenate([zW, rowconv(0), zW], axis=0)
    cp1p = jnp.concatenate([zW, rowconv(2), zW], axis=0)
    z = (rowconv(1)
         + jax.lax.slice(cm1p, (0, 0), (N_TOK, MID_CHUNK))
         + jax.lax.slice(cp1p, (2 * W, 0), (2 * W + N_TOK, MID_CHUNK)))
    z = _gelu(z + bmid_ref[1:2, :])
    part = jax.lax.dot_general(z.astype(_BF16), w2_ref[...], dn_cc,
                               preferred_element_type=_F32)

    @pl.when(nc == 0)
    def _():
        out_ref[0] = x + b3_ref[0:1, :] + part

    @pl.when(nc != 0)
    def _():
        out_ref[0] = out_ref[0] + part


@jax.jit
def kernel(x0, bn0_g, bn0_b, q_w, k_w, v_w, merge_w, merge_b,
           mlp_w1, mlp_bn1_g, mlp_bn1_b, mlp_dw, mlp_bn2_g, mlp_bn2_b,
           mlp_w2, mlp_bn3_g, mlp_bn3_b):
    inv = 1.0 / math.sqrt(1.0 + EPS)
    x0t = x0.reshape(B, D_MODEL, N_TOK).transpose(0, 2, 1)

    qw2 = (q_w * (1.0 / math.sqrt(D_HEAD))).astype(_BF16)
    kw2 = k_w.astype(_BF16)
    vw2 = v_w.astype(_BF16)
    mw2 = merge_w.astype(_BF16)

    vec1 = jnp.zeros((8, D_MODEL), _F32)
    vec1 = vec1.at[0].set(bn0_g * inv).at[1].set(bn0_b).at[2].set(merge_b)

    w1f = (mlp_w1 * (mlp_bn1_g * inv)[:, None]).astype(_BF16)
    w2f = (mlp_w2 * (mlp_bn3_g * inv)[:, None]).astype(_BF16)
    dwt = jnp.zeros((16, D_MID), _F32)
    dwt = dwt.at[:9].set((mlp_dw.reshape(D_MID, 9)
                          * (mlp_bn2_g * inv)[:, None]).T)
    bmid = jnp.zeros((8, D_MID), _F32)
    bmid = bmid.at[0].set(mlp_bn1_b).at[1].set(mlp_bn2_b)
    b3 = jnp.zeros((8, D_MODEL), _F32)
    b3 = b3.at[0].set(mlp_bn3_b)

    xmid = pl.pallas_call(
        _attn_body,
        grid=(B,),
        in_specs=[
            pl.BlockSpec((1, N_TOK, D_MODEL), lambda b: (b, 0, 0)),
            pl.BlockSpec((D_MODEL, D_MODEL), lambda b: (0, 0)),
            pl.BlockSpec((D_MODEL, D_MODEL), lambda b: (0, 0)),
            pl.BlockSpec((D_MODEL, D_MODEL), lambda b: (0, 0)),
            pl.BlockSpec((D_MODEL, D_MODEL), lambda b: (0, 0)),
            pl.BlockSpec((8, D_MODEL), lambda b: (0, 0)),
        ],
        out_specs=pl.BlockSpec((1, N_TOK, D_MODEL), lambda b: (b, 0, 0)),
        out_shape=jax.ShapeDtypeStruct((B, N_TOK, D_MODEL), _F32),
        compiler_params=pltpu.CompilerParams(
            dimension_semantics=("parallel",)),
    )(x0t, qw2, kw2, vw2, mw2, vec1)

    out = pl.pallas_call(
        _mlp_body,
        grid=(B, N_CHUNK),
        in_specs=[
            pl.BlockSpec((1, N_TOK, D_MODEL), lambda b, nc: (b, 0, 0)),
            pl.BlockSpec((MID_CHUNK, D_MODEL), lambda b, nc: (nc, 0)),
            pl.BlockSpec((D_MODEL, MID_CHUNK), lambda b, nc: (0, nc)),
            pl.BlockSpec((16, MID_CHUNK), lambda b, nc: (0, nc)),
            pl.BlockSpec((8, MID_CHUNK), lambda b, nc: (0, nc)),
            pl.BlockSpec((8, D_MODEL), lambda b, nc: (0, 0)),
        ],
        out_specs=pl.BlockSpec((1, N_TOK, D_MODEL), lambda b, nc: (b, 0, 0)),
        out_shape=jax.ShapeDtypeStruct((B, N_TOK, D_MODEL), _F32),
        compiler_params=pltpu.CompilerParams(
            dimension_semantics=("parallel", "arbitrary")),
    )(xmid, w1f, w2f, dwt, bmid, b3)

    return out.transpose(0, 2, 1).reshape(B, D_MODEL, H, W)


# speedup vs baseline: 1.6108x; 1.0162x over previous
import math

import jax
import jax.numpy as jnp
from jax.experimental import pallas as pl
from jax.experimental.pallas import tpu as pltpu

D_MODEL = 384
D_HEAD = 64
N_HEAD = D_MODEL // D_HEAD
D_MID = D_MODEL * 4
B, H, W = 4, 32, 32
N_TOK = H * W
EPS = 1e-5
PAD = 40
MID_CHUNK = 1536
N_CHUNK = D_MID // MID_CHUNK

_F32 = jnp.float32
_BF16 = jnp.bfloat16


def _gelu(x):
    return 0.5 * x * (1.0 + jax.lax.erf(x * (1.0 / math.sqrt(2.0))))


def _attn_body(x0_ref, qw_ref, kw_ref, vw_ref, mw_ref, vec_ref, out_ref):
    x0 = x0_ref[0]
    sc0 = vec_ref[0:1, :]
    b0 = vec_ref[1:2, :]
    mb = vec_ref[2:3, :]
    xg = _gelu(x0 * sc0 + b0).astype(_BF16)
    ones = jnp.ones((N_TOK, D_HEAD), _BF16)
    dn_cc = (((1,), (1,)), ((), ()))
    qa = jax.lax.dot_general(xg, qw_ref[...], dn_cc,
                             preferred_element_type=_F32).astype(_BF16)
    ka = jax.lax.dot_general(xg, kw_ref[...], dn_cc,
                             preferred_element_type=_F32).astype(_BF16)
    va = jax.lax.dot_general(xg, vw_ref[...], dn_cc,
                             preferred_element_type=_F32).astype(_BF16)
    mhs = []
    for h in range(N_HEAD):
        qh = jax.lax.slice(qa, (0, h * D_HEAD), (N_TOK, (h + 1) * D_HEAD))
        kh = jax.lax.slice(ka, (0, h * D_HEAD), (N_TOK, (h + 1) * D_HEAD))
        vh = jax.lax.slice(va, (0, h * D_HEAD), (N_TOK, (h + 1) * D_HEAD))
        s = jax.lax.dot_general(qh, kh, dn_cc, preferred_element_type=_F32)
        p = jnp.exp(jnp.minimum(s, 40.0)).astype(_BF16)
        vext = jnp.concatenate([vh, ones], axis=1)
        mv = jax.lax.dot_general(p, vext, (((1,), (0,)), ((), ())),
                                 preferred_element_type=_F32)
        l = jax.lax.slice(mv, (0, D_HEAD), (N_TOK, D_HEAD + 1))
        mhs.append((jax.lax.slice(mv, (0, 0), (N_TOK, D_HEAD)) / l).astype(_BF16))
    msg = jnp.concatenate(mhs, axis=1)
    out_ref[0] = x0 + mb + jax.lax.dot_general(
        msg, mw_ref[...], dn_cc, preferred_element_type=_F32)


def _mlp_body(x_ref, w1_ref, w2_ref, dwt_ref, bmid_ref, b3_ref, out_ref):
    nc = pl.program_id(1)
    x = x_ref[0]
    dn_cc = (((1,), (1,)), ((), ()))
    y = _gelu(jax.lax.dot_general(x.astype(_BF16), w1_ref[...], dn_cc,
                                  preferred_element_type=_F32)
              + bmid_ref[0:1, :])
    z8 = jnp.zeros((8, MID_CHUNK), _F32)
    yp8 = jnp.concatenate([z8, y, z8], axis=0)
    col = jax.lax.broadcasted_iota(jnp.int32, (N_TOK, 1), 0) % W
    um = jnp.where(col >= 1,
                   jax.lax.slice(yp8, (7, 0), (7 + N_TOK, MID_CHUNK)), 0.0)
    up = jnp.where(col <= W - 2,
                   jax.lax.slice(yp8, (9, 0), (9 + N_TOK, MID_CHUNK)), 0.0)

    def rowconv(i):
        return (um * dwt_ref[3 * i:3 * i + 1, :]
                + y * dwt_ref[3 * i + 1:3 * i + 2, :]
                + up * dwt_ref[3 * i + 2:3 * i + 3, :])

    zW = jnp.zeros((W, MID_CHUNK), _F32)
    cm1p = jnp.concatenate([zW, rowconv(0), zW], axis=0)
    cp1p = jnp.concatenate([zW, rowconv(2), zW], axis=0)
    z = (rowconv(1)
         + jax.lax.slice(cm1p, (0, 0), (N_TOK, MID_CHUNK))
         + jax.lax.slice(cp1p, (2 * W, 0), (2 * W + N_TOK, MID_CHUNK)))
    z = _gelu(z + bmid_ref[1:2, :])
    part = jax.lax.dot_general(z.astype(_BF16), w2_ref[...], dn_cc,
                               preferred_element_type=_F32)

    @pl.when(nc == 0)
    def _():
        out_ref[0] = x + b3_ref[0:1, :] + part

    @pl.when(nc != 0)
    def _():
        out_ref[0] = out_ref[0] + part


@jax.jit
def kernel(x0, bn0_g, bn0_b, q_w, k_w, v_w, merge_w, merge_b,
           mlp_w1, mlp_bn1_g, mlp_bn1_b, mlp_dw, mlp_bn2_g, mlp_bn2_b,
           mlp_w2, mlp_bn3_g, mlp_bn3_b):
    inv = 1.0 / math.sqrt(1.0 + EPS)
    x0t = x0.reshape(B, D_MODEL, N_TOK).transpose(0, 2, 1)

    qw2 = (q_w * (1.0 / math.sqrt(D_HEAD))).astype(_BF16)
    kw2 = k_w.astype(_BF16)
    vw2 = v_w.astype(_BF16)
    mw2 = merge_w.astype(_BF16)

    vec1 = jnp.zeros((8, D_MODEL), _F32)
    vec1 = vec1.at[0].set(bn0_g * inv).at[1].set(bn0_b).at[2].set(merge_b)

    w1f = (mlp_w1 * (mlp_bn1_g * inv)[:, None]).astype(_BF16)
    w2f = (mlp_w2 * (mlp_bn3_g * inv)[:, None]).astype(_BF16)
    dwt = jnp.zeros((16, D_MID), _F32)
    dwt = dwt.at[:9].set((mlp_dw.reshape(D_MID, 9)
                          * (mlp_bn2_g * inv)[:, None]).T)
    bmid = jnp.zeros((8, D_MID), _F32)
    bmid = bmid.at[0].set(mlp_bn1_b).at[1].set(mlp_bn2_b)
    b3 = jnp.zeros((8, D_MODEL), _F32)
    b3 = b3.at[0].set(mlp_bn3_b)

    xmid = pl.pallas_call(
        _attn_body,
        grid=(B,),
        in_specs=[
            pl.BlockSpec((1, N_TOK, D_MODEL), lambda b: (b, 0, 0)),
            pl.BlockSpec((D_MODEL, D_MODEL), lambda b: (0, 0)),
            pl.BlockSpec((D_MODEL, D_MODEL), lambda b: (0, 0)),
            pl.BlockSpec((D_MODEL, D_MODEL), lambda b: (0, 0)),
            pl.BlockSpec((D_MODEL, D_MODEL), lambda b: (0, 0)),
            pl.BlockSpec((8, D_MODEL), lambda b: (0, 0)),
        ],
        out_specs=pl.BlockSpec((1, N_TOK, D_MODEL), lambda b: (b, 0, 0)),
        out_shape=jax.ShapeDtypeStruct((B, N_TOK, D_MODEL), _F32),
        compiler_params=pltpu.CompilerParams(
            dimension_semantics=("parallel",)),
    )(x0t, qw2, kw2, vw2, mw2, vec1)

    out = pl.pallas_call(
        _mlp_body,
        grid=(B, N_CHUNK),
        in_specs=[
            pl.BlockSpec((1, N_TOK, D_MODEL), lambda b, nc: (b, 0, 0)),
            pl.BlockSpec((MID_CHUNK, D_MODEL), lambda b, nc: (nc, 0)),
            pl.BlockSpec((D_MODEL, MID_CHUNK), lambda b, nc: (0, nc)),
            pl.BlockSpec((16, MID_CHUNK), lambda b, nc: (0, nc)),
            pl.BlockSpec((8, MID_CHUNK), lambda b, nc: (0, nc)),
            pl.BlockSpec((8, D_MODEL), lambda b, nc: (0, 0)),
        ],
        out_specs=pl.BlockSpec((1, N_TOK, D_MODEL), lambda b, nc: (b, 0, 0)),
        out_shape=jax.ShapeDtypeStruct((B, N_TOK, D_MODEL), _F32),
        compiler_params=pltpu.CompilerParams(
            dimension_semantics=("parallel", "arbitrary")),
    )(xmid, w1f, w2f, dwt, bmid, b3)

    return out.transpose(0, 2, 1).reshape(B, D_MODEL, H, W)


# speedup vs baseline: 1.6550x; 1.0274x over previous
import math

import jax
import jax.numpy as jnp
from jax.experimental import pallas as pl
from jax.experimental.pallas import tpu as pltpu

D_MODEL = 384
D_HEAD = 64
N_HEAD = D_MODEL // D_HEAD
D_MID = D_MODEL * 4
B, H, W = 4, 32, 32
N_TOK = H * W
EPS = 1e-5
PAD = 40
MID_CHUNK = 1536
N_CHUNK = D_MID // MID_CHUNK

_F32 = jnp.float32
_BF16 = jnp.bfloat16


def _gelu(x):
    return 0.5 * x * (1.0 + jax.lax.erf(x * (1.0 / math.sqrt(2.0))))


def _block_body(x0_ref, qw_ref, kw_ref, vw_ref, mw_ref, vec_ref,
                w1_ref, w2_ref, dwt_ref, bmid_ref, b3_ref, out_ref):
    x0 = x0_ref[0]
    sc0 = vec_ref[0:1, :]
    b0 = vec_ref[1:2, :]
    mb = vec_ref[2:3, :]
    xg = _gelu(x0 * sc0 + b0).astype(_BF16)
    ones = jnp.ones((N_TOK, D_HEAD), _BF16)
    dn_cc = (((1,), (1,)), ((), ()))
    qa = jax.lax.dot_general(xg, qw_ref[...], dn_cc,
                             preferred_element_type=_F32).astype(_BF16)
    ka = jax.lax.dot_general(xg, kw_ref[...], dn_cc,
                             preferred_element_type=_F32).astype(_BF16)
    va = jax.lax.dot_general(xg, vw_ref[...], dn_cc,
                             preferred_element_type=_F32).astype(_BF16)
    mhs = []
    for h in range(N_HEAD):
        qh = jax.lax.slice(qa, (0, h * D_HEAD), (N_TOK, (h + 1) * D_HEAD))
        kh = jax.lax.slice(ka, (0, h * D_HEAD), (N_TOK, (h + 1) * D_HEAD))
        vh = jax.lax.slice(va, (0, h * D_HEAD), (N_TOK, (h + 1) * D_HEAD))
        s = jax.lax.dot_general(qh, kh, dn_cc, preferred_element_type=_F32)
        p = jnp.exp(jnp.minimum(s, 40.0)).astype(_BF16)
        vext = jnp.concatenate([vh, ones], axis=1)
        mv = jax.lax.dot_general(p, vext, (((1,), (0,)), ((), ())),
                                 preferred_element_type=_F32)
        l = jax.lax.slice(mv, (0, D_HEAD), (N_TOK, D_HEAD + 1))
        mhs.append((jax.lax.slice(mv, (0, 0), (N_TOK, D_HEAD)) / l).astype(_BF16))
    msg = jnp.concatenate(mhs, axis=1)
    x = x0 + mb + jax.lax.dot_general(
        msg, mw_ref[...], dn_cc, preferred_element_type=_F32)
    y = _gelu(jax.lax.dot_general(x.astype(_BF16), w1_ref[...], dn_cc,
                                  preferred_element_type=_F32)
              + bmid_ref[0:1, :])
    z8 = jnp.zeros((8, MID_CHUNK), _F32)
    yp8 = jnp.concatenate([z8, y, z8], axis=0)
    col = jax.lax.broadcasted_iota(jnp.int32, (N_TOK, 1), 0) % W
    um = jnp.where(col >= 1,
                   jax.lax.slice(yp8, (7, 0), (7 + N_TOK, MID_CHUNK)), 0.0)
    up = jnp.where(col <= W - 2,
                   jax.lax.slice(yp8, (9, 0), (9 + N_TOK, MID_CHUNK)), 0.0)

    def rowconv(i):
        return (um * dwt_ref[3 * i:3 * i + 1, :]
                + y * dwt_ref[3 * i + 1:3 * i + 2, :]
                + up * dwt_ref[3 * i + 2:3 * i + 3, :])

    zW = jnp.zeros((W, MID_CHUNK), _F32)
    cm1p = jnp.concatenate([zW, rowconv(0), zW], axis=0)
    cp1p = jnp.concatenate([zW, rowconv(2), zW], axis=0)
    z = (rowconv(1)
         + jax.lax.slice(cm1p, (0, 0), (N_TOK, MID_CHUNK))
         + jax.lax.slice(cp1p, (2 * W, 0), (2 * W + N_TOK, MID_CHUNK)))
    z = _gelu(z + bmid_ref[1:2, :])
    part = jax.lax.dot_general(z.astype(_BF16), w2_ref[...], dn_cc,
                               preferred_element_type=_F32)
    out_ref[0] = x + b3_ref[0:1, :] + part


@jax.jit
def kernel(x0, bn0_g, bn0_b, q_w, k_w, v_w, merge_w, merge_b,
           mlp_w1, mlp_bn1_g, mlp_bn1_b, mlp_dw, mlp_bn2_g, mlp_bn2_b,
           mlp_w2, mlp_bn3_g, mlp_bn3_b):
    inv = 1.0 / math.sqrt(1.0 + EPS)
    x0t = x0.reshape(B, D_MODEL, N_TOK).transpose(0, 2, 1)

    qw2 = (q_w * (1.0 / math.sqrt(D_HEAD))).astype(_BF16)
    kw2 = k_w.astype(_BF16)
    vw2 = v_w.astype(_BF16)
    mw2 = merge_w.astype(_BF16)

    vec1 = jnp.zeros((8, D_MODEL), _F32)
    vec1 = vec1.at[0].set(bn0_g * inv).at[1].set(bn0_b).at[2].set(merge_b)

    w1f = (mlp_w1 * (mlp_bn1_g * inv)[:, None]).astype(_BF16)
    w2f = (mlp_w2 * (mlp_bn3_g * inv)[:, None]).astype(_BF16)
    dwt = jnp.zeros((16, D_MID), _F32)
    dwt = dwt.at[:9].set((mlp_dw.reshape(D_MID, 9)
                          * (mlp_bn2_g * inv)[:, None]).T)
    bmid = jnp.zeros((8, D_MID), _F32)
    bmid = bmid.at[0].set(mlp_bn1_b).at[1].set(mlp_bn2_b)
    b3 = jnp.zeros((8, D_MODEL), _F32)
    b3 = b3.at[0].set(mlp_bn3_b)

    out = pl.pallas_call(
        _block_body,
        grid=(B,),
        in_specs=[
            pl.BlockSpec((1, N_TOK, D_MODEL), lambda b: (b, 0, 0)),
            pl.BlockSpec((D_MODEL, D_MODEL), lambda b: (0, 0)),
            pl.BlockSpec((D_MODEL, D_MODEL), lambda b: (0, 0)),
            pl.BlockSpec((D_MODEL, D_MODEL), lambda b: (0, 0)),
            pl.BlockSpec((D_MODEL, D_MODEL), lambda b: (0, 0)),
            pl.BlockSpec((8, D_MODEL), lambda b: (0, 0)),
            pl.BlockSpec((D_MID, D_MODEL), lambda b: (0, 0)),
            pl.BlockSpec((D_MODEL, D_MID), lambda b: (0, 0)),
            pl.BlockSpec((16, D_MID), lambda b: (0, 0)),
            pl.BlockSpec((8, D_MID), lambda b: (0, 0)),
            pl.BlockSpec((8, D_MODEL), lambda b: (0, 0)),
        ],
        out_specs=pl.BlockSpec((1, N_TOK, D_MODEL), lambda b: (b, 0, 0)),
        out_shape=jax.ShapeDtypeStruct((B, N_TOK, D_MODEL), _F32),
        compiler_params=pltpu.CompilerParams(
            dimension_semantics=("parallel",)),
    )(x0t, qw2, kw2, vw2, mw2, vec1, w1f, w2f, dwt, bmid, b3)

    return out.transpose(0, 2, 1).reshape(B, D_MODEL, H, W)
